# baseline (speedup 1.0000x reference)
import functools

import jax
import jax.numpy as jnp
from jax import lax
from jax.experimental import pallas as pl
from jax.experimental.pallas import tpu as pltpu
from jax.experimental.pallas import tpu_sc as plsc

B = 16384
D = 32
NC = 2
NS = 16
NW = NC * NS
BPW = B // NW
CHUNK = 128
NCHUNK = BPW // CHUNK


def _sc_gather_body(u_tab, i_tab, u_idx, i_idx, u_out, i_out,
                    idx_u_v, idx_i_v, rows_u_v, rows_i_v, sem_u, sem_i):
    wid = lax.axis_index("s") * NC + lax.axis_index("c")
    base = wid * BPW
    pltpu.sync_copy(u_idx.at[pl.ds(wid * NCHUNK, NCHUNK)], idx_u_v)
    pltpu.sync_copy(i_idx.at[pl.ds(wid * NCHUNK, NCHUNK)], idx_i_v)
    copies = []
    for j in range(NCHUNK):
        copies.append(pltpu.async_copy(
            u_tab.at[idx_u_v.at[j]], rows_u_v.at[pl.ds(j * CHUNK, CHUNK)], sem_u))
        copies.append(pltpu.async_copy(
            i_tab.at[idx_i_v.at[j]], rows_i_v.at[pl.ds(j * CHUNK, CHUNK)], sem_i))
    for c in copies:
        c.wait()
    pltpu.sync_copy(rows_u_v, u_out.at[pl.ds(base, BPW)])
    pltpu.sync_copy(rows_i_v, i_out.at[pl.ds(base, BPW)])


_sc_gather = pl.kernel(
    _sc_gather_body,
    out_type=(jax.ShapeDtypeStruct((B, D), jnp.float32),
              jax.ShapeDtypeStruct((B, D), jnp.float32)),
    mesh=plsc.VectorSubcoreMesh(core_axis_name="c", subcore_axis_name="s",
                                num_cores=NC, num_subcores=NS),
    scratch_types=[
        pltpu.VMEM((NCHUNK, CHUNK), jnp.int32),
        pltpu.VMEM((NCHUNK, CHUNK), jnp.int32),
        pltpu.VMEM((BPW, D), jnp.float32),
        pltpu.VMEM((BPW, D), jnp.float32),
        pltpu.SemaphoreType.DMA,
        pltpu.SemaphoreType.DMA,
    ],
    compiler_params=pltpu.CompilerParams(use_tc_tiling_on_sc=False),
)


BB = 2048


def _dense_body(gu, gi, uf, itf, wuf, wif, buf, bif, wu, wi, bo, out):
    u = gu[...] + jnp.dot(uf[...], wuf[...],
                          preferred_element_type=jnp.float32) + buf[...]
    v = gi[...] + jnp.dot(itf[...], wif[...],
                          preferred_element_type=jnp.float32) + bif[...]
    out[...] = (jnp.sum(u * wu[...], axis=1, keepdims=True)
                + jnp.sum(v * wi[...], axis=1, keepdims=True) + bo[...])


_dense = pl.pallas_call(
    _dense_body,
    grid=(B // BB,),
    in_specs=[
        pl.BlockSpec((BB, D), lambda i: (i, 0)),
        pl.BlockSpec((BB, D), lambda i: (i, 0)),
        pl.BlockSpec((BB, 16), lambda i: (i, 0)),
        pl.BlockSpec((BB, 16), lambda i: (i, 0)),
        pl.BlockSpec((16, D), lambda i: (0, 0)),
        pl.BlockSpec((16, D), lambda i: (0, 0)),
        pl.BlockSpec((1, D), lambda i: (0, 0)),
        pl.BlockSpec((1, D), lambda i: (0, 0)),
        pl.BlockSpec((1, D), lambda i: (0, 0)),
        pl.BlockSpec((1, D), lambda i: (0, 0)),
        pl.BlockSpec((1, 1), lambda i: (0, 0)),
    ],
    out_specs=pl.BlockSpec((BB, 1), lambda i: (i, 0)),
    out_shape=jax.ShapeDtypeStruct((B, 1), jnp.float32),
)


def kernel(user_ids, item_ids, user_features, item_features, user_emb,
           item_emb, W_uf, b_uf, W_if, b_if, W_out, b_out):
    uidx = user_ids.reshape(NW * NCHUNK, CHUNK)
    iidx = item_ids.reshape(NW * NCHUNK, CHUNK)
    gu, gi = _sc_gather(user_emb, item_emb, uidx, iidx)
    wu = W_out[:D].reshape(1, D)
    wi = W_out[D:].reshape(1, D)
    return _dense(gu, gi, user_features, item_features, W_uf, W_if,
                  b_uf.reshape(1, D), b_if.reshape(1, D), wu, wi,
                  b_out.reshape(1, 1))
